# baseline (speedup 1.0000x reference)
import jax
import jax.numpy as jnp
from jax import lax
from jax.experimental import pallas as pl
from jax.experimental.pallas import tpu as pltpu
from jax.experimental.pallas import tpu_sc as plsc

B = 16384
L = 50
D = 64
NW = 32
HALF = NW // 2
GPW = B // HALF
CPW = GPW // 2
IBC = 128
NBLK = CPW // IBC
NBUF = 4
LANES = 16
NV = D // LANES
INV_L = 1.0 / L


def _pool_body(q_hbm, c_hbm, table_hbm, qout_hbm, cout_hbm,
               idx_v, r0, r1, r2, r3, outblk, s0, s1, s2, s3, sob):
    wid = lax.axis_index("s") * 2 + lax.axis_index("c")
    rows = (r0, r1, r2, r3)
    sems = (s0, s1, s2, s3)

    def process(idx2, outref, w):
        cbase = w * CPW
        obase = w * GPW

        def block_body(blk):
            @pl.when(blk > 0)
            def _():
                pltpu.make_async_copy(
                    outblk, outref.at[pl.ds(obase + (blk - 1) * IBC * 2, IBC * 2)],
                    sob).wait()

            crow0 = cbase + blk * IBC
            pltpu.sync_copy(idx2.at[pl.ds(crow0, IBC)], idx_v)
            for s in range(NBUF):
                pltpu.async_copy(table_hbm.at[idx_v.at[s]], rows[s], sems[s])

            def quad(p):
                for s in range(NBUF):
                    q = NBUF * p + s
                    pltpu.make_async_copy(
                        table_hbm.at[idx_v.at[q]], rows[s], sems[s]).wait()
                    for g in range(2):
                        rbase = g * L
                        orow = 2 * q + g
                        for j in range(NV):
                            a = rows[s][rbase, pl.ds(j * LANES, LANES)]
                            for r in range(1, L):
                                a = a + rows[s][rbase + r, pl.ds(j * LANES, LANES)]
                            outblk[orow, pl.ds(j * LANES, LANES)] = a * INV_L

                    @pl.when(q + NBUF < IBC)
                    def _():
                        pltpu.async_copy(
                            table_hbm.at[idx_v.at[q + NBUF]], rows[s], sems[s])

            pl.loop(0, IBC // NBUF)(quad)
            pltpu.async_copy(
                outblk, outref.at[pl.ds(obase + blk * IBC * 2, IBC * 2)], sob)

        pl.loop(0, NBLK)(block_body)
        pltpu.make_async_copy(
            outblk, outref.at[pl.ds(obase + (NBLK - 1) * IBC * 2, IBC * 2)],
            sob).wait()

    @pl.when(wid < HALF)
    def _():
        process(q_hbm, qout_hbm, wid)

    @pl.when(wid >= HALF)
    def _():
        process(c_hbm, cout_hbm, wid - HALF)


@jax.jit
def _pooled_lookup(q_idx, c_idx, table):
    mesh = plsc.VectorSubcoreMesh(core_axis_name="c", subcore_axis_name="s")
    pooled_t = jax.ShapeDtypeStruct((B, D), jnp.float32)
    return pl.kernel(
        _pool_body,
        out_type=(pooled_t, pooled_t),
        mesh=mesh,
        scratch_types=[
            pltpu.VMEM((IBC, 2 * L), jnp.int32),
            pltpu.VMEM((2 * L, D), jnp.float32),
            pltpu.VMEM((2 * L, D), jnp.float32),
            pltpu.VMEM((2 * L, D), jnp.float32),
            pltpu.VMEM((2 * L, D), jnp.float32),
            pltpu.VMEM((2 * IBC, D), jnp.float32),
            pltpu.SemaphoreType.DMA,
            pltpu.SemaphoreType.DMA,
            pltpu.SemaphoreType.DMA,
            pltpu.SemaphoreType.DMA,
            pltpu.SemaphoreType.DMA,
        ],
        compiler_params=pltpu.CompilerParams(use_tc_tiling_on_sc=False),
    )(q_idx, c_idx, table)


def _mm_body(q_ref, c_ref, w_ref, b_ref, qo_ref, co_ref):
    w = w_ref[...]
    bb = b_ref[...]
    qo_ref[...] = jnp.dot(q_ref[...], w, preferred_element_type=jnp.float32) + bb
    co_ref[...] = jnp.dot(c_ref[...], w, preferred_element_type=jnp.float32) + bb


@jax.jit
def _project(q_pooled, c_pooled, W, b):
    blk = 4096
    out_t = jax.ShapeDtypeStruct((B, D), jnp.float32)
    return pl.pallas_call(
        _mm_body,
        grid=(B // blk,),
        in_specs=[
            pl.BlockSpec((blk, D), lambda i: (i, 0)),
            pl.BlockSpec((blk, D), lambda i: (i, 0)),
            pl.BlockSpec((D, D), lambda i: (0, 0)),
            pl.BlockSpec((1, D), lambda i: (0, 0)),
        ],
        out_specs=[
            pl.BlockSpec((blk, D), lambda i: (i, 0)),
            pl.BlockSpec((blk, D), lambda i: (i, 0)),
        ],
        out_shape=(out_t, out_t),
    )(q_pooled, c_pooled, W, b.reshape(1, D))


def kernel(query, candidate, table, W, b):
    q_idx = query.astype(jnp.int32).reshape(B // 2, 2 * L)
    c_idx = candidate.astype(jnp.int32).reshape(B // 2, 2 * L)
    q_pooled, c_pooled = _pooled_lookup(q_idx, c_idx, table)
    q_out, c_out = _project(q_pooled, c_pooled, W, b)
    return (q_out, c_out)


# speedup vs baseline: 1.1688x; 1.1688x over previous
import jax
import jax.numpy as jnp
from jax import lax
from jax.experimental import pallas as pl
from jax.experimental.pallas import tpu as pltpu
from jax.experimental.pallas import tpu_sc as plsc

B = 16384
L = 50
D = 64
NG = 2 * B
NW = 32
GPW = NG // NW
IB = 64
NB = GPW // IB
NBUF = 4
LANES = 16
NV = D // LANES
INV_L = 1.0 / L


def _pool_body(idx_hbm, table_hbm, out_hbm,
               idx_v, r0, r1, r2, r3, outblk, s0, s1, s2, s3, sob):
    wid = lax.axis_index("s") * 2 + lax.axis_index("c")
    base = wid * GPW
    rows = (r0, r1, r2, r3)
    sems = (s0, s1, s2, s3)

    def accumulate(buf, g):
        for j in range(NV):
            a = buf[0, pl.ds(j * LANES, LANES)]
            for r in range(1, L):
                a = a + buf[r, pl.ds(j * LANES, LANES)]
            outblk[g, pl.ds(j * LANES, LANES)] = a * INV_L

    def block_body(blk):
        @pl.when(blk > 0)
        def _():
            pltpu.make_async_copy(
                outblk, out_hbm.at[pl.ds(base + (blk - 1) * IB, IB)], sob).wait()

        row0 = base + blk * IB
        pltpu.sync_copy(idx_hbm.at[pl.ds(row0, IB)], idx_v)
        for s in range(NBUF):
            pltpu.async_copy(table_hbm.at[idx_v.at[s]], rows[s], sems[s])

        def quad(p):
            for s in range(NBUF):
                g = NBUF * p + s
                pltpu.make_async_copy(
                    table_hbm.at[idx_v.at[g]], rows[s], sems[s]).wait()
                accumulate(rows[s], g)

                @pl.when(g + NBUF < IB)
                def _():
                    pltpu.async_copy(
                        table_hbm.at[idx_v.at[g + NBUF]], rows[s], sems[s])

        pl.loop(0, IB // NBUF)(quad)
        pltpu.async_copy(outblk, out_hbm.at[pl.ds(row0, IB)], sob)

    pl.loop(0, NB)(block_body)
    pltpu.make_async_copy(
        outblk, out_hbm.at[pl.ds(base + (NB - 1) * IB, IB)], sob).wait()


@jax.jit
def _pooled_lookup(idx, table):
    mesh = plsc.VectorSubcoreMesh(core_axis_name="c", subcore_axis_name="s")
    return pl.kernel(
        _pool_body,
        out_type=jax.ShapeDtypeStruct((NG, D), jnp.float32),
        mesh=mesh,
        scratch_types=[
            pltpu.VMEM((IB, L), jnp.int32),
            pltpu.VMEM((L, D), jnp.float32),
            pltpu.VMEM((L, D), jnp.float32),
            pltpu.VMEM((L, D), jnp.float32),
            pltpu.VMEM((L, D), jnp.float32),
            pltpu.VMEM((IB, D), jnp.float32),
            pltpu.SemaphoreType.DMA,
            pltpu.SemaphoreType.DMA,
            pltpu.SemaphoreType.DMA,
            pltpu.SemaphoreType.DMA,
            pltpu.SemaphoreType.DMA,
        ],
        compiler_params=pltpu.CompilerParams(use_tc_tiling_on_sc=False),
    )(idx, table)


def _mm_body(x_ref, w_ref, b_ref, o_ref):
    o_ref[...] = (
        jnp.dot(x_ref[...], w_ref[...], preferred_element_type=jnp.float32)
        + b_ref[...]
    )


@jax.jit
def _project(pooled, W, b):
    blk = 4096
    return pl.pallas_call(
        _mm_body,
        grid=(NG // blk,),
        in_specs=[
            pl.BlockSpec((blk, D), lambda i: (i, 0)),
            pl.BlockSpec((D, D), lambda i: (0, 0)),
            pl.BlockSpec((1, D), lambda i: (0, 0)),
        ],
        out_specs=pl.BlockSpec((blk, D), lambda i: (i, 0)),
        out_shape=jax.ShapeDtypeStruct((NG, D), jnp.float32),
    )(pooled, W, b.reshape(1, D))


def kernel(query, candidate, table, W, b):
    idx = jnp.concatenate([query, candidate], axis=0).astype(jnp.int32)
    pooled = _pooled_lookup(idx, table)
    out = _project(pooled, W, b)
    return (out[:B], out[B:])


# speedup vs baseline: 1.5991x; 1.3681x over previous
import jax
import jax.numpy as jnp
from jax import lax
from jax.experimental import pallas as pl
from jax.experimental.pallas import tpu as pltpu
from jax.experimental.pallas import tpu_sc as plsc

B = 16384
L = 50
D = 64
NG = 2 * B
NW = 32
GPW = NG // NW
IB = 64
NB = GPW // IB
NBUF = 4
LANES = 16
NV = D // LANES
INV_L = 1.0 / L


def _pool_body(idx_hbm, table_hbm, out_hbm,
               idx_v, r0, r1, r2, r3, outblk, s0, s1, s2, s3, sob):
    wid = lax.axis_index("s") * 2 + lax.axis_index("c")
    base = wid * GPW
    rows = (r0, r1, r2, r3)
    sems = (s0, s1, s2, s3)

    def accumulate(buf, g):
        def rbody(r, accs):
            return tuple(
                accs[j] + buf[r, pl.ds(j * LANES, LANES)] for j in range(NV)
            )
        init = tuple(buf[0, pl.ds(j * LANES, LANES)] for j in range(NV))
        accs = lax.fori_loop(1, L, rbody, init, unroll=5)
        for j in range(NV):
            outblk[g, pl.ds(j * LANES, LANES)] = accs[j] * INV_L

    def block_body(blk):
        @pl.when(blk > 0)
        def _():
            pltpu.make_async_copy(
                outblk, out_hbm.at[pl.ds(base + (blk - 1) * IB, IB)], sob).wait()

        row0 = base + blk * IB
        pltpu.sync_copy(idx_hbm.at[pl.ds(row0, IB)], idx_v)
        for s in range(NBUF):
            pltpu.async_copy(table_hbm.at[idx_v.at[s]], rows[s], sems[s])

        def quad(p):
            for s in range(NBUF):
                g = NBUF * p + s
                pltpu.make_async_copy(
                    table_hbm.at[idx_v.at[g]], rows[s], sems[s]).wait()
                accumulate(rows[s], g)

                @pl.when(g + NBUF < IB)
                def _():
                    pltpu.async_copy(
                        table_hbm.at[idx_v.at[g + NBUF]], rows[s], sems[s])

        pl.loop(0, IB // NBUF)(quad)
        pltpu.async_copy(outblk, out_hbm.at[pl.ds(row0, IB)], sob)

    pl.loop(0, NB)(block_body)
    pltpu.make_async_copy(
        outblk, out_hbm.at[pl.ds(base + (NB - 1) * IB, IB)], sob).wait()


@jax.jit
def _pooled_lookup(idx, table):
    mesh = plsc.VectorSubcoreMesh(core_axis_name="c", subcore_axis_name="s")
    return pl.kernel(
        _pool_body,
        out_type=jax.ShapeDtypeStruct((NG, D), jnp.float32),
        mesh=mesh,
        scratch_types=[
            pltpu.VMEM((IB, L), jnp.int32),
            pltpu.VMEM((L, D), jnp.float32),
            pltpu.VMEM((L, D), jnp.float32),
            pltpu.VMEM((L, D), jnp.float32),
            pltpu.VMEM((L, D), jnp.float32),
            pltpu.VMEM((IB, D), jnp.float32),
            pltpu.SemaphoreType.DMA,
            pltpu.SemaphoreType.DMA,
            pltpu.SemaphoreType.DMA,
            pltpu.SemaphoreType.DMA,
            pltpu.SemaphoreType.DMA,
        ],
        compiler_params=pltpu.CompilerParams(use_tc_tiling_on_sc=False),
    )(idx, table)


def _mm_body(x_ref, w_ref, b_ref, o_ref):
    o_ref[...] = (
        jnp.dot(x_ref[...], w_ref[...], preferred_element_type=jnp.float32)
        + b_ref[...]
    )


@jax.jit
def _project(pooled, W, b):
    blk = 4096
    return pl.pallas_call(
        _mm_body,
        grid=(NG // blk,),
        in_specs=[
            pl.BlockSpec((blk, D), lambda i: (i, 0)),
            pl.BlockSpec((D, D), lambda i: (0, 0)),
            pl.BlockSpec((1, D), lambda i: (0, 0)),
        ],
        out_specs=pl.BlockSpec((blk, D), lambda i: (i, 0)),
        out_shape=jax.ShapeDtypeStruct((NG, D), jnp.float32),
    )(pooled, W, b.reshape(1, D))


def kernel(query, candidate, table, W, b):
    idx = jnp.concatenate([query, candidate], axis=0).astype(jnp.int32)
    pooled = _pooled_lookup(idx, table)
    out = _project(pooled, W, b)
    return (out[:B], out[B:])
